# baseline (speedup 1.0000x reference)
import functools

import jax
import jax.numpy as jnp
from jax import lax
from jax.experimental import pallas as pl
from jax.experimental.pallas import tpu as pltpu
from jax.experimental.pallas import tpu_sc as plsc

N = 10000
E = 320000
D = 128
L = 16
NT = 16
NC = 2
NW = NC * NT
NPAD = 10240
ROWS = NPAD // L
ROWS_PER_TILE = ROWS // NT
EDGES_PER_TILE = E // NW
EVECS_PER_TILE = EDGES_PER_TILE // L
XROWS_PER_TILE = 312
XTAIL = N - NW * XROWS_PER_TILE
XCHUNK = 78
NCHUNKS = XROWS_PER_TILE // XCHUNK
IDX_CHUNK = 128


def _zero_rows(ref, nrows):
    zeros = jnp.zeros((L,), jnp.float32)

    @plsc.parallel_loop(0, nrows, unroll=8)
    def _(i):
        ref[i] = zeros


def _gnn_body(x_hbm, edge_hbm, attr_hbm,
              out_hbm, hpart_hbm, flags_hbm, parts_hbm,
              src_v, dst_v, attr_v, w_v, wnew_v, pbuf_v, xbuf0, xbuf1,
              idx_v, acc_v, part_v, zbuf, fbuf_v, w_sh, sem):
    cid = lax.axis_index("c")
    sid = lax.axis_index("s")
    wid = cid * NT + sid

    ebase = wid * EDGES_PER_TILE
    pltpu.async_copy(edge_hbm.at[0, pl.ds(ebase, EDGES_PER_TILE)],
                     src_v, sem)
    pltpu.async_copy(attr_hbm.at[pl.ds(ebase, EDGES_PER_TILE)],
                     attr_v, sem)
    pltpu.async_copy(edge_hbm.at[1, pl.ds(ebase, EDGES_PER_TILE)],
                     dst_v, sem)

    for j in range(ROWS // IDX_CHUNK):
        for k in range(IDX_CHUNK // L):
            idx_v[j, pl.ds(k * L, L)] = (
                lax.iota(jnp.int32, L) + (j * IDX_CHUNK + k * L))
    _zero_rows(zbuf, ROWS_PER_TILE)
    _zero_rows(wnew_v, ROWS)

    pltpu.make_async_copy(edge_hbm.at[0, pl.ds(ebase, EDGES_PER_TILE)],
                          src_v, sem).wait()
    pltpu.make_async_copy(attr_hbm.at[pl.ds(ebase, EDGES_PER_TILE)],
                          attr_v, sem).wait()

    def edge_pass(first):
        @plsc.parallel_loop(0, EVECS_PER_TILE, unroll=5)
        def _(i):
            s = src_v[pl.ds(i * L, L)]
            a = attr_v[pl.ds(i * L, L)]
            if first:
                m = a
            else:
                d = dst_v[pl.ds(i * L, L)]
                wd = plsc.load_gather(
                    w_v, [lax.shift_right_logical(d, 4),
                          jnp.bitwise_and(d, 15)])
                m = wd * a
            plsc.addupdate_scatter(
                wnew_v, [lax.shift_right_logical(s, 4),
                         jnp.bitwise_and(s, 15)], m)

    def flag_row(core, p):
        return flags_hbm.at[core * 4 + p]

    def publish_and_merge(p):
        @pl.when(sid == 0)
        def _():
            pltpu.sync_copy(w_sh, hpart_hbm.at[cid, p])
            fbuf_v[pl.ds(0, L)] = jnp.full((L,), p + 1, jnp.int32)
            pltpu.sync_copy(fbuf_v, flag_row(cid, p))

            def poll_body(_):
                pltpu.sync_copy(flag_row(1 - cid, p), fbuf_v)
                v = fbuf_v[pl.ds(0, L)]
                return jnp.logical_and(jnp.min(v) == p + 1,
                                       jnp.max(v) == p + 1)

            lax.while_loop(lambda done: jnp.logical_not(done),
                           poll_body, jnp.bool_(False))
            fbuf_v[pl.ds(0, L)] = jnp.zeros((L,), jnp.int32)
            pltpu.sync_copy(fbuf_v, flag_row(1 - cid, p))

            pltpu.sync_copy(hpart_hbm.at[1 - cid, p], pbuf_v)
            for j in range(ROWS // IDX_CHUNK):
                pltpu.sync_copy(pbuf_v.at[pl.ds(j * IDX_CHUNK, IDX_CHUNK)],
                                w_sh.at[idx_v.at[j]], add=True)

    node0 = wid * XROWS_PER_TILE
    bufs = [xbuf0, xbuf1]

    def x_slice(c):
        return x_hbm.at[pl.ds(node0 + c * XCHUNK, XCHUNK)]

    for p in range(3):
        edge_pass(first=(p == 0))
        if p == 0:
            pltpu.make_async_copy(edge_hbm.at[1, pl.ds(ebase,
                                                       EDGES_PER_TILE)],
                                  dst_v, sem).wait()

        pltpu.sync_copy(zbuf, w_sh.at[pl.ds(sid * ROWS_PER_TILE,
                                            ROWS_PER_TILE)])
        plsc.subcore_barrier()
        nj = ROWS // IDX_CHUNK
        for jj in range(nj):
            j = lax.rem(sid + jj, nj)
            pltpu.sync_copy(wnew_v.at[pl.ds(j * IDX_CHUNK, IDX_CHUNK)],
                            w_sh.at[idx_v.at[j]], add=True)
        if p < 2:
            _zero_rows(wnew_v, ROWS)
        plsc.subcore_barrier()

        publish_and_merge(p)
        plsc.subcore_barrier()
        if p == 2:
            pltpu.async_copy(x_slice(0), bufs[0], sem)
            pltpu.async_copy(x_slice(1), bufs[1], sem)
        pltpu.sync_copy(w_sh, w_v)
        plsc.subcore_barrier()

    acc = tuple(jnp.zeros((L,), jnp.float32) for _ in range(D // L))
    for c in range(NCHUNKS):
        buf = bufs[c % 2]
        pltpu.make_async_copy(x_slice(c), buf, sem).wait()

        @plsc.parallel_loop(0, XCHUNK, unroll=6, carry=acc)
        def row_body(r, acc):
            ln = node0 + c * XCHUNK + r
            wi = plsc.load_gather(
                w_v, [jnp.full((L,), lax.shift_right_logical(ln, 4),
                               jnp.int32),
                      jnp.full((L,), jnp.bitwise_and(ln, 15), jnp.int32)])
            return tuple(acc[k] + wi * buf[r, pl.ds(k * L, L)]
                         for k in range(D // L))

        acc = row_body
        if c + 2 < NCHUNKS:
            pltpu.async_copy(x_slice(c + 2), bufs[c % 2], sem)

    def tail_fn(acc):
        pltpu.sync_copy(x_hbm.at[pl.ds(NW * XROWS_PER_TILE, XTAIL)],
                        xbuf0.at[pl.ds(0, XTAIL)])

        @plsc.parallel_loop(0, XTAIL, unroll=4, carry=acc)
        def tail_body(r, acc):
            ln = NW * XROWS_PER_TILE + r
            wi = plsc.load_gather(
                w_v, [jnp.full((L,), lax.shift_right_logical(ln, 4),
                               jnp.int32),
                      jnp.full((L,), jnp.bitwise_and(ln, 15), jnp.int32)])
            return tuple(acc[k] + wi * xbuf0[r, pl.ds(k * L, L)]
                         for k in range(D // L))

        return tail_body

    acc = lax.cond(wid == NW - 1, tail_fn, lambda a: a, acc)

    scale = jnp.float32(1.0 / N)
    for k in range(D // L):
        acc_v[0, pl.ds(k * L, L)] = acc[k] * scale
    pltpu.sync_copy(acc_v, parts_hbm.at[pl.ds(wid, 1)])
    plsc.subcore_barrier()

    @pl.when(jnp.logical_and(cid == 1, sid == 0))
    def _():
        fbuf_v[pl.ds(0, L)] = jnp.full((L,), 4, jnp.int32)
        pltpu.sync_copy(fbuf_v, flag_row(1, 3))

    @pl.when(jnp.logical_and(cid == 0, sid == 0))
    def _():
        def poll_body(_):
            pltpu.sync_copy(flag_row(1, 3), fbuf_v)
            v = fbuf_v[pl.ds(0, L)]
            return jnp.logical_and(jnp.min(v) == 4, jnp.max(v) == 4)

        lax.while_loop(lambda done: jnp.logical_not(done),
                       poll_body, jnp.bool_(False))
        fbuf_v[pl.ds(0, L)] = jnp.zeros((L,), jnp.int32)
        pltpu.sync_copy(fbuf_v, flag_row(1, 3))

        pltpu.sync_copy(parts_hbm, part_v)
        for k in range(D // L):
            tot = part_v[0, pl.ds(k * L, L)]
            for r in range(1, NW):
                tot = tot + part_v[r, pl.ds(k * L, L)]
            acc_v[0, pl.ds(k * L, L)] = tot
        pltpu.sync_copy(acc_v, out_hbm)


@functools.lru_cache(maxsize=1)
def _build_gnn_sc():
    return functools.partial(
        pl.kernel,
        out_type=(
            jax.ShapeDtypeStruct((1, D), jnp.float32),
            jax.ShapeDtypeStruct((NC, 3, ROWS, L), jnp.float32),
            jax.ShapeDtypeStruct((NC * 4, L), jnp.int32),
            jax.ShapeDtypeStruct((NW, D), jnp.float32),
        ),
        mesh=plsc.VectorSubcoreMesh(core_axis_name="c", subcore_axis_name="s",
                                    num_cores=NC, num_subcores=NT),
        compiler_params=pltpu.CompilerParams(use_tc_tiling_on_sc=False,
                                             needs_layout_passes=False),
        scratch_types=[
            pltpu.VMEM((EDGES_PER_TILE,), jnp.int32),
            pltpu.VMEM((EDGES_PER_TILE,), jnp.int32),
            pltpu.VMEM((EDGES_PER_TILE,), jnp.float32),
            pltpu.VMEM((ROWS, L), jnp.float32),
            pltpu.VMEM((ROWS, L), jnp.float32),
            pltpu.VMEM((ROWS, L), jnp.float32),
            pltpu.VMEM((XCHUNK, D), jnp.float32),
            pltpu.VMEM((XCHUNK, D), jnp.float32),
            pltpu.VMEM((ROWS // IDX_CHUNK, IDX_CHUNK), jnp.int32),
            pltpu.VMEM((1, D), jnp.float32),
            pltpu.VMEM((NW, D), jnp.float32),
            pltpu.VMEM((ROWS_PER_TILE, L), jnp.float32),
            pltpu.VMEM((L,), jnp.int32),
            pltpu.VMEM_SHARED((ROWS, L), jnp.float32),
            pltpu.SemaphoreType.DMA,
        ],
    )(_gnn_body)


def kernel(x, edge_index, edge_attr, batch):
    del batch
    out, _, _, _ = _build_gnn_sc()(x, edge_index, edge_attr)
    return out


# speedup vs baseline: 1.0489x; 1.0489x over previous
import functools

import jax
import jax.numpy as jnp
from jax import lax
from jax.experimental import pallas as pl
from jax.experimental.pallas import tpu as pltpu
from jax.experimental.pallas import tpu_sc as plsc

N = 10000
E = 320000
D = 128
L = 16
NT = 16
NPAD = 10240
ROWS = NPAD // L
ROWS_PER_TILE = ROWS // NT
EDGES_PER_TILE = E // NT
EVECS_PER_TILE = EDGES_PER_TILE // L
XROWS_PER_TILE = N // NT
XCHUNK = 125
NCHUNKS = XROWS_PER_TILE // XCHUNK
IDX_CHUNK = 128


def _zero_rows(ref, nrows):
    zeros = jnp.zeros((L,), jnp.float32)

    @plsc.parallel_loop(0, nrows, unroll=8)
    def _(i):
        ref[i] = zeros


def _gnn_body(x_hbm, edge_hbm, attr_hbm, out_hbm,
              src_v, dst_v, attr_v, w_v, wnew_v, xbuf0, xbuf1, idx_v, acc_v,
              part_v, zbuf, w_sh, part_sh, sem):
    cid = lax.axis_index("c")
    sid = lax.axis_index("s")

    @pl.when(cid == 0)
    def _():
        ebase = sid * EDGES_PER_TILE
        pltpu.async_copy(edge_hbm.at[0, pl.ds(ebase, EDGES_PER_TILE)],
                         src_v, sem)
        pltpu.async_copy(attr_hbm.at[pl.ds(ebase, EDGES_PER_TILE)],
                         attr_v, sem)
        pltpu.async_copy(edge_hbm.at[1, pl.ds(ebase, EDGES_PER_TILE)],
                         dst_v, sem)

        for j in range(ROWS // IDX_CHUNK):
            for k in range(IDX_CHUNK // L):
                idx_v[j, pl.ds(k * L, L)] = (
                    lax.iota(jnp.int32, L) + (j * IDX_CHUNK + k * L))

        pltpu.make_async_copy(edge_hbm.at[0, pl.ds(ebase, EDGES_PER_TILE)],
                              src_v, sem).wait()
        pltpu.make_async_copy(attr_hbm.at[pl.ds(ebase, EDGES_PER_TILE)],
                              attr_v, sem).wait()

        def edge_pass(first):
            @plsc.parallel_loop(0, EVECS_PER_TILE, unroll=4)
            def _(i):
                s = src_v[pl.ds(i * L, L)]
                a = attr_v[pl.ds(i * L, L)]
                if first:
                    m = a
                else:
                    d = dst_v[pl.ds(i * L, L)]
                    wd = plsc.load_gather(
                        w_v, [lax.shift_right_logical(d, 4),
                              jnp.bitwise_and(d, 15)])
                    m = wd * a
                plsc.addupdate_scatter(
                    wnew_v, [lax.shift_right_logical(s, 4),
                             jnp.bitwise_and(s, 15)], m)

        _zero_rows(zbuf, ROWS_PER_TILE)

        node0 = sid * XROWS_PER_TILE
        bufs = [xbuf0, xbuf1]

        def x_slice(c):
            return x_hbm.at[pl.ds(node0 + c * XCHUNK, XCHUNK)]

        _zero_rows(wnew_v, ROWS)
        for p in range(3):
            edge_pass(first=(p == 0))
            if p == 0:
                pltpu.make_async_copy(
                    edge_hbm.at[1, pl.ds(ebase, EDGES_PER_TILE)],
                    dst_v, sem).wait()

            pltpu.sync_copy(zbuf, w_sh.at[pl.ds(sid * ROWS_PER_TILE,
                                                ROWS_PER_TILE)])
            plsc.subcore_barrier()
            nj = ROWS // IDX_CHUNK
            for jj in range(nj):
                j = lax.rem(sid + jj, nj)
                pltpu.sync_copy(wnew_v.at[pl.ds(j * IDX_CHUNK, IDX_CHUNK)],
                                w_sh.at[idx_v.at[j]], add=True)
            if p < 2:
                _zero_rows(wnew_v, ROWS)
            plsc.subcore_barrier()
            if p == 2:
                pltpu.async_copy(x_slice(0), bufs[0], sem)
                pltpu.async_copy(x_slice(1), bufs[1], sem)
            pltpu.sync_copy(w_sh, w_v)
            plsc.subcore_barrier()

        acc = tuple(jnp.zeros((L,), jnp.float32) for _ in range(D // L))
        for c in range(NCHUNKS):
            buf = bufs[c % 2]
            pltpu.make_async_copy(x_slice(c), buf, sem).wait()

            @plsc.parallel_loop(0, XCHUNK, unroll=5, carry=acc)
            def row_body(r, acc):
                ln = node0 + c * XCHUNK + r
                wi = plsc.load_gather(
                    w_v, [jnp.full((L,), lax.shift_right_logical(ln, 4),
                                   jnp.int32),
                          jnp.full((L,), jnp.bitwise_and(ln, 15), jnp.int32)])
                return tuple(acc[k] + wi * buf[r, pl.ds(k * L, L)]
                             for k in range(D // L))

            acc = row_body
            if c + 2 < NCHUNKS:
                pltpu.async_copy(x_slice(c + 2), bufs[c % 2], sem)

        scale = jnp.float32(1.0 / N)
        for k in range(D // L):
            acc_v[0, pl.ds(k * L, L)] = acc[k] * scale
        pltpu.sync_copy(acc_v, part_sh.at[pl.ds(sid, 1)])
        plsc.subcore_barrier()

        @pl.when(sid == 0)
        def _():
            pltpu.sync_copy(part_sh, part_v)
            for k in range(D // L):
                tot = part_v[0, pl.ds(k * L, L)]
                for r in range(1, NT):
                    tot = tot + part_v[r, pl.ds(k * L, L)]
                acc_v[0, pl.ds(k * L, L)] = tot
            pltpu.sync_copy(acc_v, out_hbm)


@functools.lru_cache(maxsize=1)
def _build_gnn_sc():
    return functools.partial(
        pl.kernel,
        out_type=jax.ShapeDtypeStruct((1, D), jnp.float32),
        mesh=plsc.VectorSubcoreMesh(core_axis_name="c", subcore_axis_name="s",
                                    num_cores=2, num_subcores=NT),
        compiler_params=pltpu.CompilerParams(use_tc_tiling_on_sc=False,
                                             needs_layout_passes=False),
        scratch_types=[
            pltpu.VMEM((EDGES_PER_TILE,), jnp.int32),
            pltpu.VMEM((EDGES_PER_TILE,), jnp.int32),
            pltpu.VMEM((EDGES_PER_TILE,), jnp.float32),
            pltpu.VMEM((ROWS, L), jnp.float32),
            pltpu.VMEM((ROWS, L), jnp.float32),
            pltpu.VMEM((XCHUNK, D), jnp.float32),
            pltpu.VMEM((XCHUNK, D), jnp.float32),
            pltpu.VMEM((ROWS // IDX_CHUNK, IDX_CHUNK), jnp.int32),
            pltpu.VMEM((1, D), jnp.float32),
            pltpu.VMEM((NT, D), jnp.float32),
            pltpu.VMEM((ROWS_PER_TILE, L), jnp.float32),
            pltpu.VMEM_SHARED((ROWS, L), jnp.float32),
            pltpu.VMEM_SHARED((NT, D), jnp.float32),
            pltpu.SemaphoreType.DMA,
        ],
    )(_gnn_body)


def kernel(x, edge_index, edge_attr, batch):
    del batch
    return _build_gnn_sc()(x, edge_index, edge_attr)
